# baseline (speedup 1.0000x reference)
import functools

import jax
import jax.numpy as jnp
from jax import lax
from jax.experimental import pallas as pl
from jax.experimental.pallas import tpu as pltpu
from jax.experimental.pallas import tpu_sc as plsc

N = 10000
E = 320000
D = 128
G = 64
OUT = 5

NC = 2
NS = 16
NW = NC * NS
EPAD = 327680
EPW = EPAD // NW
CH = 128
NCH = EPW // CH
GRPC = 16
NGRP = NCH // GRPC
GEDG = GRPC * CH
NBUF = 2
NPT = 624
NTAIL = N - NPT * NS
ND = N + 8
ZROWS = 16

RPW = 312
PCH = 64
PNCH = 6

_mesh = plsc.VectorSubcoreMesh(
    core_axis_name="c", subcore_axis_name="s", num_cores=NC, num_subcores=NS)


@functools.partial(
    pl.kernel,
    out_type=jax.ShapeDtypeStruct((NC, N, D), jnp.float32),
    mesh=_mesh,
    scratch_types=[
        pltpu.VMEM((GRPC, CH), jnp.int32),
        pltpu.VMEM((GRPC, CH), jnp.int32),
        pltpu.VMEM((NBUF, CH, D), jnp.float32),
        pltpu.VMEM_SHARED((ND, D), jnp.float32),
    ] + [pltpu.SemaphoreType.DMA] * (2 * NBUF + 1),
)
def _sc_scatter(x_hbm, src_hbm, dst_hbm, out_hbm,
                sidx_v, didx_v, rows_v, agg_sh, *sems):
    gsems = sems[:NBUF]
    ssems = sems[NBUF:2 * NBUF]
    zsem = sems[2 * NBUF]
    c = lax.axis_index("c")
    s = lax.axis_index("s")
    w = s * NC + c

    @pl.loop(0, CH)
    def _(i):
        for cc in range(D // 16):
            rows_v[0, i, pl.ds(cc * 16, 16)] = jnp.zeros((16,), jnp.float32)

    ZC = NPT // CH
    ZT = NPT - ZC * CH

    for k in range(ZC):
        pltpu.async_copy(rows_v.at[0], agg_sh.at[pl.ds(s * NPT + k * CH, CH)],
                         zsem)
    pltpu.async_copy(rows_v.at[0, pl.ds(0, ZT)],
                     agg_sh.at[pl.ds(s * NPT + ZC * CH, ZT)], zsem)

    @pl.when(s == 0)
    def _():
        pltpu.async_copy(rows_v.at[0, pl.ds(0, NTAIL)],
                         agg_sh.at[pl.ds(NPT * NS, NTAIL)], zsem)

    for k in range(ZC):
        pltpu.make_async_copy(
            rows_v.at[0], agg_sh.at[pl.ds(s * NPT, CH)], zsem).wait()
    pltpu.make_async_copy(rows_v.at[0, pl.ds(0, ZT)],
                          agg_sh.at[pl.ds(s * NPT, ZT)], zsem).wait()

    @pl.when(s == 0)
    def _():
        pltpu.make_async_copy(
            rows_v.at[0, pl.ds(0, NTAIL)],
            agg_sh.at[pl.ds(NPT * NS, NTAIL)], zsem).wait()

    plsc.subcore_barrier()

    def gather_start(jj, b):
        pltpu.async_copy(x_hbm.at[sidx_v.at[jj]], rows_v.at[b], gsems[b])

    def gather_wait(jj, b):
        pltpu.make_async_copy(
            x_hbm.at[sidx_v.at[jj]], rows_v.at[b], gsems[b]).wait()

    def scatter_start(jj, b):
        pltpu.async_copy(rows_v.at[b], agg_sh.at[didx_v.at[jj]], ssems[b],
                         add=True)

    def scatter_wait(jj, b):
        pltpu.make_async_copy(
            rows_v.at[b], agg_sh.at[didx_v.at[jj]], ssems[b]).wait()

    @pl.loop(0, NGRP)
    def _(g):
        base = (w * NGRP + g) * GRPC
        pltpu.sync_copy(src_hbm.at[pl.ds(base, GRPC)], sidx_v)
        pltpu.sync_copy(dst_hbm.at[pl.ds(base, GRPC)], didx_v)
        gather_start(0, 0)

        @pl.loop(0, GRPC // NBUF)
        def _(i):
            for b in range(NBUF):
                jj = i * NBUF + b
                gather_wait(jj, b)
                scatter_start(jj, b)
                bn = 1 - b
                if b == 0:
                    @pl.when(jj >= 1)
                    def _():
                        scatter_wait(jj - 1, bn)
                    gather_start(jj + 1, bn)
                else:
                    scatter_wait(jj - 1, bn)

                    @pl.when(jj + 1 < GRPC)
                    def _():
                        gather_start(jj + 1, bn)

        scatter_wait(GRPC - 1, (GRPC - 1) % NBUF)

    plsc.subcore_barrier()
    pltpu.sync_copy(agg_sh.at[pl.ds(s * NPT, NPT)],
                    out_hbm.at[c, pl.ds(s * NPT, NPT)])

    @pl.when(s == 0)
    def _():
        pltpu.sync_copy(agg_sh.at[pl.ds(NPT * NS, NTAIL)],
                        out_hbm.at[c, pl.ds(NPT * NS, NTAIL)])


def _mlp_body(x_ref, a_ref, w1_ref, b1_ref, g_ref, be_ref, w2_ref, b2_ref,
              out_ref):
    h = x_ref[...] + a_ref[0] + a_ref[1]
    z = lax.dot_general(h, w1_ref[...], (((1,), (1,)), ((), ())),
                        preferred_element_type=jnp.float32)
    cbn = 1.0 / jnp.sqrt(jnp.float32(1.0 + 1e-5))
    scale = g_ref[...] * cbn
    z = z * scale + (b1_ref[...] * scale + be_ref[...])
    z = jnp.maximum(z, 0.0)
    z = lax.dot_general(z, w2_ref[...], (((1,), (1,)), ((), ())),
                        preferred_element_type=jnp.float32)
    out_ref[...] = jnp.maximum(z + b2_ref[...], 0.0)


_MLP_ROWS = 1000

_tc_mlp = pl.pallas_call(
    _mlp_body,
    grid=(N // _MLP_ROWS,),
    in_specs=[
        pl.BlockSpec((_MLP_ROWS, D), lambda i: (i, 0)),
        pl.BlockSpec((NC, _MLP_ROWS, D), lambda i: (0, i, 0)),
        pl.BlockSpec((D, D), lambda i: (0, 0)),
        pl.BlockSpec((1, D), lambda i: (0, 0)),
        pl.BlockSpec((1, D), lambda i: (0, 0)),
        pl.BlockSpec((1, D), lambda i: (0, 0)),
        pl.BlockSpec((D, D), lambda i: (0, 0)),
        pl.BlockSpec((1, D), lambda i: (0, 0)),
    ],
    out_specs=pl.BlockSpec((_MLP_ROWS, D), lambda i: (i, 0)),
    out_shape=jax.ShapeDtypeStruct((N, D), jnp.float32),
)


def _mlp_pool_body(s_ref, x_ref, a_ref, w1_ref, b1_ref, g_ref, be_ref,
                   w2_ref, b2_ref, wl_ref, bl_ref, out_ref,
                   acc_ref):
    i = pl.program_id(0)

    @pl.when(i == 0)
    def _():
        acc_ref[...] = jnp.full((G, D), -jnp.inf, jnp.float32)

    h = x_ref[...] + a_ref[0] + a_ref[1]
    z = lax.dot_general(h, w1_ref[...], (((1,), (1,)), ((), ())),
                        preferred_element_type=jnp.float32)
    cbn = 1.0 / jnp.sqrt(jnp.float32(1.0 + 1e-5))
    scale = g_ref[...] * cbn
    z = z * scale + (b1_ref[...] * scale + be_ref[...])
    z = jnp.maximum(z, 0.0)
    z = lax.dot_general(z, w2_ref[...], (((1,), (1,)), ((), ())),
                        preferred_element_type=jnp.float32)
    z = jnp.maximum(z + b2_ref[...], 0.0)

    base = i * _MLP_ROWS
    lo = s_ref[base]
    hi = s_ref[base + _MLP_ROWS - 1]
    rowid = lax.broadcasted_iota(jnp.int32, (_MLP_ROWS, D), 0)

    def lower_bound(gval):
        def bstep(_, st):
            l, h = st
            mid = (l + h) // 2
            v = s_ref[base + mid]
            return (jnp.where(v < gval, mid + 1, l),
                    jnp.where(v < gval, h, mid))

        return lax.fori_loop(0, 10, bstep, (jnp.int32(0),
                                            jnp.int32(_MLP_ROWS)))[0]

    def seg(gi, seg_start):
        seg_end = lower_bound(gi + 1)
        m = (rowid >= seg_start) & (rowid < seg_end)
        sm = jnp.max(jnp.where(m, z, -jnp.inf), axis=0, keepdims=True)
        acc_ref[pl.ds(gi, 1), :] = jnp.maximum(acc_ref[pl.ds(gi, 1), :], sm)
        return seg_end

    lax.fori_loop(lo, hi + 1, seg, jnp.int32(0))

    @pl.when(i == N // _MLP_ROWS - 1)
    def _():
        out_ref[...] = lax.dot_general(
            acc_ref[...], wl_ref[...], (((1,), (1,)), ((), ())),
            preferred_element_type=jnp.float32) + bl_ref[...]


_tc_mlp_pool = pl.pallas_call(
    _mlp_pool_body,
    grid_spec=pltpu.PrefetchScalarGridSpec(
        num_scalar_prefetch=1,
        grid=(N // _MLP_ROWS,),
        in_specs=[
            pl.BlockSpec((_MLP_ROWS, D), lambda i, s: (i, 0)),
            pl.BlockSpec((NC, _MLP_ROWS, D), lambda i, s: (0, i, 0)),
            pl.BlockSpec((D, D), lambda i, s: (0, 0)),
            pl.BlockSpec((1, D), lambda i, s: (0, 0)),
            pl.BlockSpec((1, D), lambda i, s: (0, 0)),
            pl.BlockSpec((1, D), lambda i, s: (0, 0)),
            pl.BlockSpec((D, D), lambda i, s: (0, 0)),
            pl.BlockSpec((1, D), lambda i, s: (0, 0)),
            pl.BlockSpec((OUT, D), lambda i, s: (0, 0)),
            pl.BlockSpec((1, OUT), lambda i, s: (0, 0)),
        ],
        out_specs=pl.BlockSpec((G, OUT), lambda i, s: (0, 0)),
        scratch_shapes=[pltpu.VMEM((G, D), jnp.float32)],
    ),
    out_shape=jax.ShapeDtypeStruct((G, OUT), jnp.float32),
)


def kernel(x, edge_index, batch, W1_0, b1_0, gamma_0, beta_0, W2_0, b2_0,
           W1_1, b1_1, gamma_1, beta_1, W2_1, b2_1, Wlin, blin):
    src_f = jnp.concatenate(
        [edge_index[0], jnp.zeros((EPAD - E,), jnp.int32)]).reshape(
            EPAD // CH, CH)
    dst_f = jnp.concatenate(
        [edge_index[1], jnp.full((EPAD - E,), N, jnp.int32)]).reshape(
            EPAD // CH, CH)

    agg = _sc_scatter(x, src_f, dst_f)
    h1 = _tc_mlp(x, agg, W1_0, b1_0.reshape(1, D), gamma_0.reshape(1, D),
                 beta_0.reshape(1, D), W2_0, b2_0.reshape(1, D))
    agg2 = _sc_scatter(h1, src_f, dst_f)
    return _tc_mlp_pool(
        batch, h1, agg2, W1_1, b1_1.reshape(1, D), gamma_1.reshape(1, D),
        beta_1.reshape(1, D), W2_1, b2_1.reshape(1, D),
        Wlin, blin.reshape(1, OUT))


# speedup vs baseline: 3.0693x; 3.0693x over previous
import functools

import jax
import jax.numpy as jnp
from jax import lax
from jax.experimental import pallas as pl
from jax.experimental.pallas import tpu as pltpu
from jax.experimental.pallas import tpu_sc as plsc

N = 10000
E = 320000
D = 128
G = 64
OUT = 5

NC = 2
NS = 16
NW = NC * NS
EPW = E // NW
CH = 125
NCH = EPW // CH
GRPC = 16
NGRP = NCH // GRPC
NBUF = 2
NPT = 624
NTAIL = N - NPT * NS
ND = N

RPW = 312
PCH = 64
PNCH = 6

_mesh = plsc.VectorSubcoreMesh(
    core_axis_name="c", subcore_axis_name="s", num_cores=NC, num_subcores=NS)


@functools.partial(
    pl.kernel,
    out_type=jax.ShapeDtypeStruct((NC, N, D), jnp.float32),
    mesh=_mesh,
    scratch_types=[
        pltpu.VMEM((GRPC, CH), jnp.int32),
        pltpu.VMEM((GRPC, CH), jnp.int32),
        pltpu.VMEM((NBUF, CH, D), jnp.float32),
        pltpu.VMEM_SHARED((ND, D), jnp.float32),
    ] + [pltpu.SemaphoreType.DMA] * (2 * NBUF + 1),
)
def _sc_scatter(x_hbm, src_hbm, dst_hbm, out_hbm,
                sidx_v, didx_v, rows_v, agg_sh, *sems):
    gsems = sems[:NBUF]
    ssems = sems[NBUF:2 * NBUF]
    zsem = sems[2 * NBUF]
    c = lax.axis_index("c")
    s = lax.axis_index("s")
    w = s * NC + c

    @pl.loop(0, CH)
    def _(i):
        for cc in range(D // 16):
            rows_v[0, i, pl.ds(cc * 16, 16)] = jnp.zeros((16,), jnp.float32)

    ZC = NPT // CH
    ZT = NPT - ZC * CH

    for k in range(ZC):
        pltpu.async_copy(rows_v.at[0], agg_sh.at[pl.ds(s * NPT + k * CH, CH)],
                         zsem)
    pltpu.async_copy(rows_v.at[0, pl.ds(0, ZT)],
                     agg_sh.at[pl.ds(s * NPT + ZC * CH, ZT)], zsem)

    @pl.when(s == 0)
    def _():
        pltpu.async_copy(rows_v.at[0, pl.ds(0, NTAIL)],
                         agg_sh.at[pl.ds(NPT * NS, NTAIL)], zsem)

    for k in range(ZC):
        pltpu.make_async_copy(
            rows_v.at[0], agg_sh.at[pl.ds(s * NPT, CH)], zsem).wait()
    pltpu.make_async_copy(rows_v.at[0, pl.ds(0, ZT)],
                          agg_sh.at[pl.ds(s * NPT, ZT)], zsem).wait()

    @pl.when(s == 0)
    def _():
        pltpu.make_async_copy(
            rows_v.at[0, pl.ds(0, NTAIL)],
            agg_sh.at[pl.ds(NPT * NS, NTAIL)], zsem).wait()

    plsc.subcore_barrier()

    def gather_start(jj, b):
        pltpu.async_copy(x_hbm.at[sidx_v.at[jj]], rows_v.at[b], gsems[b])

    def gather_wait(jj, b):
        pltpu.make_async_copy(
            x_hbm.at[sidx_v.at[jj]], rows_v.at[b], gsems[b]).wait()

    def scatter_start(jj, b):
        pltpu.async_copy(rows_v.at[b], agg_sh.at[didx_v.at[jj]], ssems[b],
                         add=True)

    def scatter_wait(jj, b):
        pltpu.make_async_copy(
            rows_v.at[b], agg_sh.at[didx_v.at[jj]], ssems[b]).wait()

    @pl.loop(0, NGRP)
    def _(g):
        base = (w * NGRP + g) * GRPC
        pltpu.sync_copy(src_hbm.at[pl.ds(base, GRPC)], sidx_v)
        pltpu.sync_copy(dst_hbm.at[pl.ds(base, GRPC)], didx_v)
        gather_start(0, 0)

        @pl.loop(0, GRPC // NBUF)
        def _(i):
            for b in range(NBUF):
                jj = i * NBUF + b
                gather_wait(jj, b)
                scatter_start(jj, b)
                bn = 1 - b
                if b == 0:
                    @pl.when(jj >= 1)
                    def _():
                        scatter_wait(jj - 1, bn)
                    gather_start(jj + 1, bn)
                else:
                    scatter_wait(jj - 1, bn)

                    @pl.when(jj + 1 < GRPC)
                    def _():
                        gather_start(jj + 1, bn)

        scatter_wait(GRPC - 1, (GRPC - 1) % NBUF)

    plsc.subcore_barrier()
    pltpu.sync_copy(agg_sh.at[pl.ds(s * NPT, NPT)],
                    out_hbm.at[c, pl.ds(s * NPT, NPT)])

    @pl.when(s == 0)
    def _():
        pltpu.sync_copy(agg_sh.at[pl.ds(NPT * NS, NTAIL)],
                        out_hbm.at[c, pl.ds(NPT * NS, NTAIL)])


def _mlp_body(x_ref, a_ref, w1_ref, b1_ref, g_ref, be_ref, w2_ref, b2_ref,
              out_ref):
    h = x_ref[...] + a_ref[0] + a_ref[1]
    z = lax.dot_general(h, w1_ref[...], (((1,), (1,)), ((), ())),
                        preferred_element_type=jnp.float32)
    cbn = 1.0 / jnp.sqrt(jnp.float32(1.0 + 1e-5))
    scale = g_ref[...] * cbn
    z = z * scale + (b1_ref[...] * scale + be_ref[...])
    z = jnp.maximum(z, 0.0)
    z = lax.dot_general(z, w2_ref[...], (((1,), (1,)), ((), ())),
                        preferred_element_type=jnp.float32)
    out_ref[...] = jnp.maximum(z + b2_ref[...], 0.0)


_MLP_ROWS = 1000

_tc_mlp = pl.pallas_call(
    _mlp_body,
    grid=(N // _MLP_ROWS,),
    in_specs=[
        pl.BlockSpec((_MLP_ROWS, D), lambda i: (i, 0)),
        pl.BlockSpec((NC, _MLP_ROWS, D), lambda i: (0, i, 0)),
        pl.BlockSpec((D, D), lambda i: (0, 0)),
        pl.BlockSpec((1, D), lambda i: (0, 0)),
        pl.BlockSpec((1, D), lambda i: (0, 0)),
        pl.BlockSpec((1, D), lambda i: (0, 0)),
        pl.BlockSpec((D, D), lambda i: (0, 0)),
        pl.BlockSpec((1, D), lambda i: (0, 0)),
    ],
    out_specs=pl.BlockSpec((_MLP_ROWS, D), lambda i: (i, 0)),
    out_shape=jax.ShapeDtypeStruct((N, D), jnp.float32),
)


def _mlp_pool_body(s_ref, x_ref, a_ref, w1_ref, b1_ref, g_ref, be_ref,
                   w2_ref, b2_ref, wl_ref, bl_ref, out_ref,
                   acc_ref):
    i = pl.program_id(0)

    @pl.when(i == 0)
    def _():
        acc_ref[...] = jnp.full((G, D), -jnp.inf, jnp.float32)

    h = x_ref[...] + a_ref[0] + a_ref[1]
    z = lax.dot_general(h, w1_ref[...], (((1,), (1,)), ((), ())),
                        preferred_element_type=jnp.float32)
    cbn = 1.0 / jnp.sqrt(jnp.float32(1.0 + 1e-5))
    scale = g_ref[...] * cbn
    z = z * scale + (b1_ref[...] * scale + be_ref[...])
    z = jnp.maximum(z, 0.0)
    z = lax.dot_general(z, w2_ref[...], (((1,), (1,)), ((), ())),
                        preferred_element_type=jnp.float32)
    z = jnp.maximum(z + b2_ref[...], 0.0)

    base = i * _MLP_ROWS
    lo = s_ref[base]
    hi = s_ref[base + _MLP_ROWS - 1]
    rowid = lax.broadcasted_iota(jnp.int32, (_MLP_ROWS, D), 0)

    def lower_bound(gval):
        def bstep(_, st):
            l, h = st
            mid = (l + h) // 2
            v = s_ref[base + mid]
            return (jnp.where(v < gval, mid + 1, l),
                    jnp.where(v < gval, h, mid))

        return lax.fori_loop(0, 10, bstep, (jnp.int32(0),
                                            jnp.int32(_MLP_ROWS)))[0]

    def seg(gi, seg_start):
        seg_end = lower_bound(gi + 1)
        m = (rowid >= seg_start) & (rowid < seg_end)
        sm = jnp.max(jnp.where(m, z, -jnp.inf), axis=0, keepdims=True)
        acc_ref[pl.ds(gi, 1), :] = jnp.maximum(acc_ref[pl.ds(gi, 1), :], sm)
        return seg_end

    lax.fori_loop(lo, hi + 1, seg, jnp.int32(0))

    @pl.when(i == N // _MLP_ROWS - 1)
    def _():
        out_ref[...] = lax.dot_general(
            acc_ref[...], wl_ref[...], (((1,), (1,)), ((), ())),
            preferred_element_type=jnp.float32) + bl_ref[...]


_tc_mlp_pool = pl.pallas_call(
    _mlp_pool_body,
    grid_spec=pltpu.PrefetchScalarGridSpec(
        num_scalar_prefetch=1,
        grid=(N // _MLP_ROWS,),
        in_specs=[
            pl.BlockSpec((_MLP_ROWS, D), lambda i, s: (i, 0)),
            pl.BlockSpec((NC, _MLP_ROWS, D), lambda i, s: (0, i, 0)),
            pl.BlockSpec((D, D), lambda i, s: (0, 0)),
            pl.BlockSpec((1, D), lambda i, s: (0, 0)),
            pl.BlockSpec((1, D), lambda i, s: (0, 0)),
            pl.BlockSpec((1, D), lambda i, s: (0, 0)),
            pl.BlockSpec((D, D), lambda i, s: (0, 0)),
            pl.BlockSpec((1, D), lambda i, s: (0, 0)),
            pl.BlockSpec((OUT, D), lambda i, s: (0, 0)),
            pl.BlockSpec((1, OUT), lambda i, s: (0, 0)),
        ],
        out_specs=pl.BlockSpec((G, OUT), lambda i, s: (0, 0)),
        scratch_shapes=[pltpu.VMEM((G, D), jnp.float32)],
    ),
    out_shape=jax.ShapeDtypeStruct((G, OUT), jnp.float32),
)


def kernel(x, edge_index, batch, W1_0, b1_0, gamma_0, beta_0, W2_0, b2_0,
           W1_1, b1_1, gamma_1, beta_1, W2_1, b2_1, Wlin, blin):
    src_f = edge_index[0].reshape(E // CH, CH)
    dst_f = edge_index[1].reshape(E // CH, CH)

    agg = _sc_scatter(x, src_f, dst_f)
    h1 = _tc_mlp(x, agg, W1_0, b1_0.reshape(1, D), gamma_0.reshape(1, D),
                 beta_0.reshape(1, D), W2_0, b2_0.reshape(1, D))
    agg2 = _sc_scatter(h1, src_f, dst_f)
    return _tc_mlp_pool(
        batch, h1, agg2, W1_1, b1_1.reshape(1, D), gamma_1.reshape(1, D),
        beta_1.reshape(1, D), W2_1, b2_1.reshape(1, D),
        Wlin, blin.reshape(1, OUT))
